# baseline (speedup 1.0000x reference)
import functools
import math

import jax
import jax.numpy as jnp
from jax import lax
from jax.experimental import pallas as pl
from jax.experimental.pallas import tpu as pltpu
from jax.experimental.pallas import tpu_sc as plsc

TK = 10
TBETA = 0.75
TEPS = 1e-05
_NEG = -3.0e38
_POS = 3.0e38
_CB = 8
_VL = 16
_CHUNK = 32


def _lane_bcast(v, lane):
    idx = jnp.full((_VL, 1), lane, jnp.int32)
    return lax.gather(
        v, idx,
        lax.GatherDimensionNumbers(offset_dims=(), collapsed_slice_dims=(0,),
                                   start_index_map=(0,)),
        (1,), mode=lax.GatherScatterMode.PROMISE_IN_BOUNDS)


def _sort16(v):
    r = plsc.sort_key_val(v, v)
    return r[0] if isinstance(r, (list, tuple)) else r


def _merge_top(b, v):
    vs = _sort16(v)
    return _sort16(jnp.maximum(b, lax.rev(vs, dimensions=(0,))))


def _merge_bot(w, v):
    vs = _sort16(v)
    return _sort16(jnp.minimum(w, lax.rev(vs, dimensions=(0,))))


def _sc_pass1(nrows, hw, n_workers=32):
    rows_per_w = nrows // n_workers
    nv = hw // _VL
    nch = nv // _CHUNK
    mesh = plsc.VectorSubcoreMesh(core_axis_name="c", subcore_axis_name="s",
                                  num_cores=2, num_subcores=16)

    @functools.partial(
        pl.kernel,
        out_type=jax.ShapeDtypeStruct((nrows, 48), jnp.float32),
        mesh=mesh,
        scratch_types=[
            pltpu.VMEM((hw,), jnp.float32),
            pltpu.VMEM((nch * _VL,), jnp.float32),
            pltpu.VMEM((nch * _VL,), jnp.float32),
            pltpu.VMEM((48,), jnp.float32),
        ],
        compiler_params=pltpu.CompilerParams(needs_layout_passes=False),
    )
    def body(x_hbm, o_hbm, buf, cmaxb, cminb, orow):
        wid = lax.axis_index("s") * 2 + lax.axis_index("c")

        def do_row(r, carry):
            row = wid * rows_per_w + r
            pltpu.sync_copy(x_hbm.at[row], buf)

            def chunk_a(ch, sacc):
                base = ch * (_CHUNK * _VL)
                v0 = buf[pl.ds(base, _VL)]
                cmax = v0
                cmin = v0
                s0 = sacc + v0
                s1 = jnp.zeros((_VL,), jnp.float32)
                for j in range(1, _CHUNK):
                    v = buf[pl.ds(base + j * _VL, _VL)]
                    if j % 2 == 0:
                        s0 = s0 + v
                    else:
                        s1 = s1 + v
                    cmax = jnp.maximum(cmax, v)
                    cmin = jnp.minimum(cmin, v)
                cmaxb[pl.ds(ch * _VL, _VL)] = cmax
                cminb[pl.ds(ch * _VL, _VL)] = cmin
                return s0 + s1

            sacc = lax.fori_loop(0, nch, chunk_a,
                                 jnp.zeros((_VL,), jnp.float32))

            def seed_m(ch, c):
                st, sb = c
                return (_merge_top(st, cmaxb[pl.ds(ch * _VL, _VL)]),
                        _merge_bot(sb, cminb[pl.ds(ch * _VL, _VL)]))

            seedt, seedb = lax.fori_loop(
                0, nch, seed_m,
                (jnp.full((_VL,), _NEG, jnp.float32),
                 jnp.full((_VL,), _POS, jnp.float32)))
            tt = _lane_bcast(seedt, 6)
            tb = _lane_bcast(seedb, 9)

            def chunk_b(ch, c):
                best, worst = c
                hit = jnp.any(
                    (cmaxb[pl.ds(ch * _VL, _VL)] >= tt)
                    | (cminb[pl.ds(ch * _VL, _VL)] <= tb))

                def scan(c2):
                    b0, w0 = c2
                    for j in range(_CHUNK):
                        v = buf[pl.ds(ch * (_CHUNK * _VL) + j * _VL, _VL)]

                        def mg(c3, vv=v):
                            return (_merge_top(c3[0], vv),
                                    _merge_bot(c3[1], vv))

                        b0, w0 = lax.cond(jnp.any((v >= tt) | (v <= tb)),
                                          mg, lambda c3: c3, (b0, w0))
                    return b0, w0

                return lax.cond(hit, scan, lambda c2: c2, (best, worst))

            best, worst = lax.fori_loop(
                0, nch, chunk_b,
                (jnp.full((_VL,), _NEG, jnp.float32),
                 jnp.full((_VL,), _POS, jnp.float32)))

            orow[pl.ds(0, _VL)] = best
            orow[pl.ds(_VL, _VL)] = worst
            orow[pl.ds(2 * _VL, _VL)] = sacc
            pltpu.sync_copy(orow, o_hbm.at[row])
            return carry

        lax.fori_loop(0, rows_per_w, do_row, jnp.int32(0))

    return body


def _fin_body(p_ref, w_ref, bi_ref, bt_ref, a_ref, b_ref, *, n_total):
    P = p_ref[...]
    b_dim, c_dim, _ = P.shape
    sums = jnp.sum(jnp.sum(P[:, :, 2 * _VL:], axis=2), axis=0)
    mean = sums / jnp.float32(n_total)

    nc = 2 * _VL
    A = jnp.abs(P[:, :, :nc] - mean[None, :, None])
    fi = (jax.lax.broadcasted_iota(jnp.int32, A.shape, 0) * nc
          + jax.lax.broadcasted_iota(jnp.int32, A.shape, 2))
    big = jnp.int32(b_dim * nc + 1)
    acc = jnp.zeros((c_dim,), jnp.float32)
    for _ in range(TK):
        m = jnp.max(jnp.max(A, axis=2), axis=0)
        sel = jnp.where(A == m[None, :, None], fi, big)
        idx = jnp.min(jnp.min(sel, axis=2), axis=0)
        A = jnp.where(fi == idx[None, :, None], jnp.float32(-1.0), A)
        acc = acc + m
    mean_topk = acc / jnp.float32(TK)

    const = 0.5 * (1.0 + (math.pi * math.log(4.0)) ** 0.5) \
        / (2.0 * math.log(n_total)) ** 0.5
    mt = (TBETA * bt_ref[0] + (1.0 - TBETA) * mean_topk) * jnp.float32(const)
    scale = 1.0 / (mt + jnp.float32(TEPS))
    a = scale * w_ref[0]
    a_ref[0] = a
    b_ref[0] = bi_ref[0] - mean * a


def _pass2_body(x_ref, a_ref, b_ref, o_ref):
    a = a_ref[0, 0]
    b = b_ref[0, 0]
    o_ref[0] = x_ref[0] * a[:, None] + b[:, None]


def kernel(x, weight, bias, biasTOPK):
    B, C, H, W = x.shape
    HW = H * W
    xr = x.reshape(B, C, HW)

    p = _sc_pass1(B * C, HW)(x.reshape(B * C, HW))
    p3 = p.reshape(B, C, 48)

    fin = functools.partial(_fin_body, n_total=B * HW)
    a, b2 = pl.pallas_call(
        fin,
        out_shape=[jax.ShapeDtypeStruct((1, C), jnp.float32),
                   jax.ShapeDtypeStruct((1, C), jnp.float32)],
    )(p3, weight.reshape(1, C), bias.reshape(1, C), biasTOPK.reshape(1, C))

    a3 = a.reshape(C // _CB, 1, _CB)
    b3 = b2.reshape(C // _CB, 1, _CB)
    out = pl.pallas_call(
        _pass2_body,
        grid=(B, C // _CB),
        in_specs=[
            pl.BlockSpec((1, _CB, HW), lambda b, c: (b, c, 0)),
            pl.BlockSpec((1, 1, _CB), lambda b, c: (c, 0, 0)),
            pl.BlockSpec((1, 1, _CB), lambda b, c: (c, 0, 0)),
        ],
        out_specs=pl.BlockSpec((1, _CB, HW), lambda b, c: (b, c, 0)),
        out_shape=jax.ShapeDtypeStruct((B, C, HW), jnp.float32),
    )(xr, a3, b3)

    return out.reshape(B, C, H, W)


# speedup vs baseline: 1.0312x; 1.0312x over previous
import functools
import math

import jax
import jax.numpy as jnp
from jax import lax
from jax.experimental import pallas as pl
from jax.experimental.pallas import tpu as pltpu
from jax.experimental.pallas import tpu_sc as plsc

TK = 10
TBETA = 0.75
TEPS = 1e-05
_NEG = -3.0e38
_POS = 3.0e38
_CB = 8
_VL = 16
_CHUNK = 32


def _lane_bcast(v, lane):
    idx = jnp.full((_VL, 1), lane, jnp.int32)
    return lax.gather(
        v, idx,
        lax.GatherDimensionNumbers(offset_dims=(), collapsed_slice_dims=(0,),
                                   start_index_map=(0,)),
        (1,), mode=lax.GatherScatterMode.PROMISE_IN_BOUNDS)


def _sort16(v):
    r = plsc.sort_key_val(v, v)
    return r[0] if isinstance(r, (list, tuple)) else r


def _merge_top(b, v):
    vs = _sort16(v)
    return _sort16(jnp.maximum(b, lax.rev(vs, dimensions=(0,))))


def _merge_bot(w, v):
    vs = _sort16(v)
    return _sort16(jnp.minimum(w, lax.rev(vs, dimensions=(0,))))


def _sc_pass1(nrows, hw, n_workers=32):
    rows_per_w = nrows // n_workers
    nv = hw // _VL
    nch = nv // _CHUNK
    mesh = plsc.VectorSubcoreMesh(core_axis_name="c", subcore_axis_name="s",
                                  num_cores=2, num_subcores=16)

    @functools.partial(
        pl.kernel,
        out_type=jax.ShapeDtypeStruct((nrows, 48), jnp.float32),
        mesh=mesh,
        scratch_types=[
            pltpu.VMEM((hw,), jnp.float32),
            pltpu.VMEM((nch * _VL,), jnp.float32),
            pltpu.VMEM((nch * _VL,), jnp.float32),
            pltpu.VMEM((48,), jnp.float32),
        ],
        compiler_params=pltpu.CompilerParams(needs_layout_passes=False),
    )
    def body(x_hbm, o_hbm, buf, cmaxb, cminb, orow):
        wid = lax.axis_index("s") * 2 + lax.axis_index("c")

        def do_row(r, carry):
            row = wid * rows_per_w + r
            pltpu.sync_copy(x_hbm.at[row], buf)

            def chunk_a(ch, sacc):
                base = ch * (_CHUNK * _VL)
                v0 = buf[pl.ds(base, _VL)]
                cmax = v0
                cmin = v0
                s0 = sacc + v0
                s1 = jnp.zeros((_VL,), jnp.float32)
                for j in range(1, _CHUNK):
                    v = buf[pl.ds(base + j * _VL, _VL)]
                    if j % 2 == 0:
                        s0 = s0 + v
                    else:
                        s1 = s1 + v
                    cmax = jnp.maximum(cmax, v)
                    cmin = jnp.minimum(cmin, v)
                cmaxb[pl.ds(ch * _VL, _VL)] = cmax
                cminb[pl.ds(ch * _VL, _VL)] = cmin
                return s0 + s1

            sacc = lax.fori_loop(0, nch, chunk_a,
                                 jnp.zeros((_VL,), jnp.float32))

            def seed_m(ch, c):
                st, sb = c
                return (_merge_top(st, cmaxb[pl.ds(ch * _VL, _VL)]),
                        _merge_bot(sb, cminb[pl.ds(ch * _VL, _VL)]))

            seedt, seedb = lax.fori_loop(
                0, nch, seed_m,
                (jnp.full((_VL,), _NEG, jnp.float32),
                 jnp.full((_VL,), _POS, jnp.float32)))
            tt = _lane_bcast(seedt, 6)
            tb = _lane_bcast(seedb, 9)

            def chunk_b(ch, c):
                best, worst = c
                hit = jnp.any(
                    (cmaxb[pl.ds(ch * _VL, _VL)] >= tt)
                    | (cminb[pl.ds(ch * _VL, _VL)] <= tb))

                def scan(c2):
                    b0, w0 = c2
                    for j in range(_CHUNK):
                        v = buf[pl.ds(ch * (_CHUNK * _VL) + j * _VL, _VL)]

                        def mg(c3, vv=v):
                            return (_merge_top(c3[0], vv),
                                    _merge_bot(c3[1], vv))

                        b0, w0 = lax.cond(jnp.any((v >= tt) | (v <= tb)),
                                          mg, lambda c3: c3, (b0, w0))
                    return b0, w0

                return lax.cond(hit, scan, lambda c2: c2, (best, worst))

            best, worst = lax.fori_loop(
                0, nch, chunk_b,
                (jnp.full((_VL,), _NEG, jnp.float32),
                 jnp.full((_VL,), _POS, jnp.float32)))

            orow[pl.ds(0, _VL)] = best
            orow[pl.ds(_VL, _VL)] = worst
            orow[pl.ds(2 * _VL, _VL)] = sacc
            pltpu.sync_copy(orow, o_hbm.at[row])
            return carry

        lax.fori_loop(0, rows_per_w, do_row, jnp.int32(0))

    return body


def _fin_body(p_ref, w_ref, bi_ref, bt_ref, a_ref, b_ref, *, n_total):
    P = p_ref[...]
    b_dim, c_dim, _ = P.shape
    sums = jnp.sum(jnp.sum(P[:, :, 2 * _VL:], axis=2), axis=0)
    mean = sums / jnp.float32(n_total)

    nc = 2 * _VL
    A = jnp.abs(P[:, :, :nc] - mean[None, :, None])
    fi = (jax.lax.broadcasted_iota(jnp.int32, A.shape, 0) * nc
          + jax.lax.broadcasted_iota(jnp.int32, A.shape, 2))
    big = jnp.int32(b_dim * nc + 1)
    acc = jnp.zeros((c_dim,), jnp.float32)
    for _ in range(TK):
        m = jnp.max(jnp.max(A, axis=2), axis=0)
        sel = jnp.where(A == m[None, :, None], fi, big)
        idx = jnp.min(jnp.min(sel, axis=2), axis=0)
        A = jnp.where(fi == idx[None, :, None], jnp.float32(-1.0), A)
        acc = acc + m
    mean_topk = acc / jnp.float32(TK)

    const = 0.5 * (1.0 + (math.pi * math.log(4.0)) ** 0.5) \
        / (2.0 * math.log(n_total)) ** 0.5
    mt = (TBETA * bt_ref[0] + (1.0 - TBETA) * mean_topk) * jnp.float32(const)
    scale = 1.0 / (mt + jnp.float32(TEPS))
    a = scale * w_ref[0]
    a_ref[0] = a
    b_ref[0] = bi_ref[0] - mean * a


def _pass2_body(x_ref, a_ref, b_ref, o_ref):
    a = a_ref[0, 0]
    b = b_ref[0, 0]
    o_ref[0] = x_ref[0] * a[:, None] + b[:, None]


def kernel(x, weight, bias, biasTOPK):
    B, C, H, W = x.shape
    HW = H * W
    xr = x.reshape(B, C, HW)

    p = _sc_pass1(B * C, HW)(x.reshape(B * C, HW))
    p3 = p.reshape(B, C, 48)

    fin = functools.partial(_fin_body, n_total=B * HW)
    a, b2 = pl.pallas_call(
        fin,
        out_shape=[jax.ShapeDtypeStruct((1, C), jnp.float32),
                   jax.ShapeDtypeStruct((1, C), jnp.float32)],
    )(p3, weight.reshape(1, C), bias.reshape(1, C), biasTOPK.reshape(1, C))

    cb2 = 32
    a3 = a.reshape(C // cb2, 1, cb2)
    b3 = b2.reshape(C // cb2, 1, cb2)
    out = pl.pallas_call(
        _pass2_body,
        grid=(B, C // cb2),
        in_specs=[
            pl.BlockSpec((1, cb2, HW), lambda b, c: (b, c, 0)),
            pl.BlockSpec((1, 1, cb2), lambda b, c: (c, 0, 0)),
            pl.BlockSpec((1, 1, cb2), lambda b, c: (c, 0, 0)),
        ],
        out_specs=pl.BlockSpec((1, cb2, HW), lambda b, c: (b, c, 0)),
        out_shape=jax.ShapeDtypeStruct((B, C, HW), jnp.float32),
    )(xr, a3, b3)

    return out.reshape(B, C, H, W)


# speedup vs baseline: 1.1148x; 1.0811x over previous
import functools
import math

import jax
import jax.numpy as jnp
from jax import lax
from jax.experimental import pallas as pl
from jax.experimental.pallas import tpu as pltpu
from jax.experimental.pallas import tpu_sc as plsc

TK = 10
TBETA = 0.75
TEPS = 1e-05
_NEG = -3.0e38
_POS = 3.0e38
_CB = 8
_VL = 16
_CHUNK = 32


def _lane_bcast(v, lane):
    idx = jnp.full((_VL, 1), lane, jnp.int32)
    return lax.gather(
        v, idx,
        lax.GatherDimensionNumbers(offset_dims=(), collapsed_slice_dims=(0,),
                                   start_index_map=(0,)),
        (1,), mode=lax.GatherScatterMode.PROMISE_IN_BOUNDS)


def _sort16(v):
    r = plsc.sort_key_val(v, v)
    return r[0] if isinstance(r, (list, tuple)) else r


def _merge_top(b, v):
    vs = _sort16(v)
    return _sort16(jnp.maximum(b, lax.rev(vs, dimensions=(0,))))


def _merge_bot(w, v):
    vs = _sort16(v)
    return _sort16(jnp.minimum(w, lax.rev(vs, dimensions=(0,))))


def _sc_pass1(nrows, hw, n_workers=32):
    rows_per_w = nrows // n_workers
    nv = hw // _VL
    nch = nv // _CHUNK
    mesh = plsc.VectorSubcoreMesh(core_axis_name="c", subcore_axis_name="s",
                                  num_cores=2, num_subcores=16)

    @functools.partial(
        pl.kernel,
        out_type=jax.ShapeDtypeStruct((nrows, 48), jnp.float32),
        mesh=mesh,
        scratch_types=[
            pltpu.VMEM((hw,), jnp.float32),
            pltpu.VMEM((hw,), jnp.float32),
            pltpu.VMEM((nch * _VL,), jnp.float32),
            pltpu.VMEM((nch * _VL,), jnp.float32),
            pltpu.VMEM((48,), jnp.float32),
            pltpu.SemaphoreType.DMA,
            pltpu.SemaphoreType.DMA,
        ],
        compiler_params=pltpu.CompilerParams(needs_layout_passes=False),
    )
    def body(x_hbm, o_hbm, bufa, bufb, cmaxb, cminb, orow, sem0, sem1):
        wid = lax.axis_index("s") * 2 + lax.axis_index("c")
        row0 = wid * rows_per_w
        sems = (sem0, sem1)
        bufs = (bufa, bufb)
        pltpu.async_copy(x_hbm.at[row0], bufa, sem0)

        def do_row(row, hb):
            buf = bufs[hb]
            nxt = jnp.minimum(row + 1, jnp.int32(nrows - 1))
            pltpu.async_copy(x_hbm.at[nxt], bufs[1 - hb], sems[1 - hb])
            pltpu.make_async_copy(x_hbm.at[row], buf, sems[hb]).wait()

            def chunk_a(ch, sacc):
                base = ch * (_CHUNK * _VL)
                v0 = buf[pl.ds(base, _VL)]
                cmax = v0
                cmin = v0
                s0 = sacc + v0
                s1 = jnp.zeros((_VL,), jnp.float32)
                for j in range(1, _CHUNK):
                    v = buf[pl.ds(base + j * _VL, _VL)]
                    if j % 2 == 0:
                        s0 = s0 + v
                    else:
                        s1 = s1 + v
                    cmax = jnp.maximum(cmax, v)
                    cmin = jnp.minimum(cmin, v)
                cmaxb[pl.ds(ch * _VL, _VL)] = cmax
                cminb[pl.ds(ch * _VL, _VL)] = cmin
                return s0 + s1

            sacc = lax.fori_loop(0, nch, chunk_a,
                                 jnp.zeros((_VL,), jnp.float32))

            def seed_m(ch, c):
                st, sb = c
                return (_merge_top(st, cmaxb[pl.ds(ch * _VL, _VL)]),
                        _merge_bot(sb, cminb[pl.ds(ch * _VL, _VL)]))

            seedt, seedb = lax.fori_loop(
                0, nch, seed_m,
                (jnp.full((_VL,), _NEG, jnp.float32),
                 jnp.full((_VL,), _POS, jnp.float32)))
            tt = _lane_bcast(seedt, 6)
            tb = _lane_bcast(seedb, 9)

            def chunk_b(ch, c):
                best, worst = c
                hit = jnp.any(
                    (cmaxb[pl.ds(ch * _VL, _VL)] >= tt)
                    | (cminb[pl.ds(ch * _VL, _VL)] <= tb))

                def scan(c2):
                    b0, w0 = c2
                    for j in range(_CHUNK):
                        v = buf[pl.ds(ch * (_CHUNK * _VL) + j * _VL, _VL)]

                        def mg(c3, vv=v):
                            return (_merge_top(c3[0], vv),
                                    _merge_bot(c3[1], vv))

                        b0, w0 = lax.cond(jnp.any((v >= tt) | (v <= tb)),
                                          mg, lambda c3: c3, (b0, w0))
                    return b0, w0

                return lax.cond(hit, scan, lambda c2: c2, (best, worst))

            best, worst = lax.fori_loop(
                0, nch, chunk_b,
                (jnp.full((_VL,), _NEG, jnp.float32),
                 jnp.full((_VL,), _POS, jnp.float32)))

            orow[pl.ds(0, _VL)] = best
            orow[pl.ds(_VL, _VL)] = worst
            orow[pl.ds(2 * _VL, _VL)] = sacc
            pltpu.sync_copy(orow, o_hbm.at[row])

        def do_pair(p, carry):
            for hb in range(2):
                do_row(row0 + p * 2 + hb, hb)
            return carry

        lax.fori_loop(0, rows_per_w // 2, do_pair, jnp.int32(0))
        pltpu.make_async_copy(x_hbm.at[row0], bufa, sem0).wait()

    return body


def _fin_body(p_ref, w_ref, bi_ref, bt_ref, a_ref, b_ref, *, n_total):
    P = p_ref[...]
    b_dim, c_dim, _ = P.shape
    sums = jnp.sum(jnp.sum(P[:, :, 2 * _VL:], axis=2), axis=0)
    mean = sums / jnp.float32(n_total)

    nc = 2 * _VL
    A = jnp.abs(P[:, :, :nc] - mean[None, :, None])
    fi = (jax.lax.broadcasted_iota(jnp.int32, A.shape, 0) * nc
          + jax.lax.broadcasted_iota(jnp.int32, A.shape, 2))
    big = jnp.int32(b_dim * nc + 1)
    acc = jnp.zeros((c_dim,), jnp.float32)
    for _ in range(TK):
        m = jnp.max(jnp.max(A, axis=2), axis=0)
        sel = jnp.where(A == m[None, :, None], fi, big)
        idx = jnp.min(jnp.min(sel, axis=2), axis=0)
        A = jnp.where(fi == idx[None, :, None], jnp.float32(-1.0), A)
        acc = acc + m
    mean_topk = acc / jnp.float32(TK)

    const = 0.5 * (1.0 + (math.pi * math.log(4.0)) ** 0.5) \
        / (2.0 * math.log(n_total)) ** 0.5
    mt = (TBETA * bt_ref[0] + (1.0 - TBETA) * mean_topk) * jnp.float32(const)
    scale = 1.0 / (mt + jnp.float32(TEPS))
    a = scale * w_ref[0]
    a_ref[0] = a
    b_ref[0] = bi_ref[0] - mean * a


def _pass2_body(x_ref, a_ref, b_ref, o_ref):
    a = a_ref[0, 0]
    b = b_ref[0, 0]
    o_ref[0] = x_ref[0] * a[:, None] + b[:, None]


def kernel(x, weight, bias, biasTOPK):
    B, C, H, W = x.shape
    HW = H * W
    xr = x.reshape(B, C, HW)

    p = _sc_pass1(B * C, HW)(x.reshape(B * C, HW))
    p3 = p.reshape(B, C, 48)

    fin = functools.partial(_fin_body, n_total=B * HW)
    a, b2 = pl.pallas_call(
        fin,
        out_shape=[jax.ShapeDtypeStruct((1, C), jnp.float32),
                   jax.ShapeDtypeStruct((1, C), jnp.float32)],
    )(p3, weight.reshape(1, C), bias.reshape(1, C), biasTOPK.reshape(1, C))

    cb2 = 32
    a3 = a.reshape(C // cb2, 1, cb2)
    b3 = b2.reshape(C // cb2, 1, cb2)
    out = pl.pallas_call(
        _pass2_body,
        grid=(B, C // cb2),
        in_specs=[
            pl.BlockSpec((1, cb2, HW), lambda b, c: (b, c, 0)),
            pl.BlockSpec((1, 1, cb2), lambda b, c: (c, 0, 0)),
            pl.BlockSpec((1, 1, cb2), lambda b, c: (c, 0, 0)),
        ],
        out_specs=pl.BlockSpec((1, cb2, HW), lambda b, c: (b, c, 0)),
        out_shape=jax.ShapeDtypeStruct((B, C, HW), jnp.float32),
    )(xr, a3, b3)

    return out.reshape(B, C, H, W)
